# baseline (speedup 1.0000x reference)
import jax
import jax.numpy as jnp
from jax.experimental import pallas as pl

_N = 10000
_E = 320000
_D = 128
_DE = 16
_H = 128
_NG = 64
_NC = 3


def _matmul_bias_kernel(xr, wr, br, outr):
    outr[...] = jnp.dot(xr[...], wr[...], preferred_element_type=jnp.float32) + br[...]


def _mm(x, w, b):
    m, k = x.shape
    n = w.shape[1]
    return pl.pallas_call(
        _matmul_bias_kernel,
        out_shape=jax.ShapeDtypeStruct((m, n), jnp.float32),
    )(x, w, b.reshape(1, n))


def _gate_kernel(zr, wmr, bmr, wgr, bgr, outr):
    z = zr[...]
    gate = jax.nn.sigmoid(jnp.dot(z, wmr[...], preferred_element_type=jnp.float32) + bmr[...])
    core = jax.nn.softplus(jnp.dot(z, wgr[...], preferred_element_type=jnp.float32) + bgr[...])
    outr[...] = gate * core


def _gate(z, wm, bm, wg, bg):
    e, k = z.shape
    blk = 8000
    return pl.pallas_call(
        _gate_kernel,
        grid=(e // blk,),
        in_specs=[
            pl.BlockSpec((blk, k), lambda i: (i, 0)),
            pl.BlockSpec((k, _H), lambda i: (0, 0)),
            pl.BlockSpec((1, _H), lambda i: (0, 0)),
            pl.BlockSpec((k, _H), lambda i: (0, 0)),
            pl.BlockSpec((1, _H), lambda i: (0, 0)),
        ],
        out_specs=pl.BlockSpec((blk, _H), lambda i: (i, 0)),
        out_shape=jax.ShapeDtypeStruct((e, _H), jnp.float32),
    )(z, wm, bm.reshape(1, _H), wg, bg.reshape(1, _H))


def kernel(x, edge_index, edge_attr, batch, W_emb, b_emb, Wm, bm, Wg, bg, W_r1, b_r1, W_r2, b_r2):
    h = _mm(x, W_emb, b_emb)
    src = edge_index[0]
    dst = edge_index[1]
    for c in range(_NC):
        z = jnp.concatenate([h[dst], h[src], edge_attr], axis=1)
        m = _gate(z, Wm[c], bm[c], Wg[c], bg[c])
        aggr = jax.ops.segment_sum(m, dst, num_segments=_N)
        h = jax.nn.softplus(h + aggr)
    sums = jax.ops.segment_sum(h, batch, num_segments=_NG)
    counts = jax.ops.segment_sum(jnp.ones((_N, 1), dtype=jnp.float32), batch, num_segments=_NG)
    pooled = sums / jnp.maximum(counts, 1.0)
    r = jax.nn.softplus(_mm(pooled, W_r1, b_r1))
    out = _mm(r, W_r2, b_r2)
    return out
